# baseline (speedup 1.0000x reference)
import jax
import jax.numpy as jnp
from jax.experimental import pallas as pl
from jax.experimental.pallas import tpu as pltpu

B, T, IN_W = 32, 256, 263
WIDTH, CODE_DIM, NB_CODE, NQ = 512, 64, 8192, 6
DOWN_T, DEPTH, DGR = 2, 3, 3

TN = 128


def _rvq_body(z_ref, cb_ref, cb2_ref, quant_ref, commit_ref, res_ref):
    q = pl.program_id(1)

    @pl.when(q == 0)
    def _():
        res_ref[...] = z_ref[...]

    residual = res_ref[...]
    cbc = cb_ref[0]
    cb2 = cb2_ref[0]

    dn = (((1,), (1,)), ((), ()))
    m = jax.lax.dot_general(residual, cbc, dn,
                            preferred_element_type=jnp.float32)

    r2 = jnp.sum(residual * residual, axis=1, keepdims=True)
    dist = r2 - 2.0 * m + cb2

    dmin = jnp.min(dist, axis=1, keepdims=True)
    iota = jax.lax.broadcasted_iota(jnp.int32, (TN, NB_CODE), 1)
    idx = jnp.min(jnp.where(dist == dmin, iota, NB_CODE), axis=1,
                  keepdims=True)
    onehot = (iota == idx).astype(jnp.bfloat16)

    c1 = cbc.astype(jnp.bfloat16)
    c1f = c1.astype(jnp.float32)
    rem1 = cbc - c1f
    c2 = rem1.astype(jnp.bfloat16)
    c3 = (rem1 - c2.astype(jnp.float32)).astype(jnp.bfloat16)
    dg = (((1,), (0,)), ((), ()))
    gv = lambda cpart: jax.lax.dot_general(
        onehot, cpart, dg, preferred_element_type=jnp.float32)
    qv = gv(c1) + gv(c2) + gv(c3)

    new_res = residual - qv
    res_ref[...] = new_res
    err = jnp.sum(new_res * new_res)

    @pl.when(q == 0)
    def _():
        quant_ref[...] = qv
        commit_ref[...] = jnp.full((1, 1, 128), err, dtype=jnp.float32)

    @pl.when(q > 0)
    def _():
        quant_ref[...] += qv
        commit_ref[...] += jnp.full((1, 1, 128), err, dtype=jnp.float32)


def _residual_vq(flat, codebooks):
    n, c = flat.shape
    n_tiles = n // TN
    cb2 = jnp.sum(codebooks * codebooks, axis=2)[:, None, :]
    quant, commit_parts = pl.pallas_call(
        _rvq_body,
        grid=(n_tiles, NQ),
        in_specs=[
            pl.BlockSpec((TN, c), lambda i, q: (i, 0)),
            pl.BlockSpec((1, NB_CODE, CODE_DIM), lambda i, q: (q, 0, 0)),
            pl.BlockSpec((1, 1, NB_CODE), lambda i, q: (q, 0, 0)),
        ],
        out_specs=[
            pl.BlockSpec((TN, c), lambda i, q: (i, 0)),
            pl.BlockSpec((1, 1, 128), lambda i, q: (i, 0, 0)),
        ],
        out_shape=[
            jax.ShapeDtypeStruct((n, c), jnp.float32),
            jax.ShapeDtypeStruct((n_tiles, 1, 128), jnp.float32),
        ],
        scratch_shapes=[pltpu.VMEM((TN, c), jnp.float32)],
    )(flat, codebooks, cb2)
    commit = jnp.sum(commit_parts[:, 0, 0]) / jnp.float32(n * c)
    return quant, commit


def _conv1d(x, w, b, stride=1, pad=0, dil=1):
    y = jax.lax.conv_general_dilated(
        x, w, (stride,), [(pad, pad)], rhs_dilation=(dil,),
        dimension_numbers=('NCH', 'OIH', 'NCH'))
    return y + b[None, :, None]


def _resblock(h, w1, b1, w2, b2, dil):
    r = jax.nn.relu(h)
    r = _conv1d(r, w1, b1, pad=dil, dil=dil)
    r = jax.nn.relu(r)
    r = _conv1d(r, w2, b2)
    return h + r


def _encode(h, p):
    h = jax.nn.relu(_conv1d(h, p['enc_in_w'], p['enc_in_b'], pad=1))
    for bl in range(DOWN_T):
        h = _conv1d(h, p['enc_down%d_w' % bl], p['enc_down%d_b' % bl],
                    stride=2, pad=1)
        for d in range(DEPTH):
            dil = DGR ** d
            h = _resblock(h, p['enc_res%d_%d_1_w' % (bl, d)],
                          p['enc_res%d_%d_1_b' % (bl, d)],
                          p['enc_res%d_%d_2_w' % (bl, d)],
                          p['enc_res%d_%d_2_b' % (bl, d)], dil)
    return _conv1d(h, p['enc_out_w'], p['enc_out_b'], pad=1)


def _decode(h, p):
    h = jax.nn.relu(_conv1d(h, p['dec_in_w'], p['dec_in_b'], pad=1))
    for bl in range(DOWN_T):
        for d in range(DEPTH):
            dil = DGR ** (DEPTH - 1 - d)
            h = _resblock(h, p['dec_res%d_%d_1_w' % (bl, d)],
                          p['dec_res%d_%d_1_b' % (bl, d)],
                          p['dec_res%d_%d_2_w' % (bl, d)],
                          p['dec_res%d_%d_2_b' % (bl, d)], dil)
        h = jnp.repeat(h, 2, axis=2)
        h = _conv1d(h, p['dec_up%d_w' % bl], p['dec_up%d_b' % bl], pad=1)
    h = jax.nn.relu(_conv1d(h, p['dec_out1_w'], p['dec_out1_b'], pad=1))
    return _conv1d(h, p['dec_out2_w'], p['dec_out2_b'], pad=1)


def kernel(x, codebooks, params):
    h = jnp.transpose(x, (0, 2, 1))
    z = _encode(h, params)
    b, c, t = z.shape
    flat = jnp.transpose(z, (0, 2, 1)).reshape(-1, c)
    quant, commit = _residual_vq(flat, codebooks)
    zq = jnp.transpose(quant.reshape(b, t, c), (0, 2, 1))
    out = _decode(zq, params)
    return jnp.transpose(out, (0, 2, 1)), commit


# speedup vs baseline: 1.0621x; 1.0621x over previous
import jax
import jax.numpy as jnp
from jax.experimental import pallas as pl
from jax.experimental.pallas import tpu as pltpu

B, T, IN_W = 32, 256, 263
WIDTH, CODE_DIM, NB_CODE, NQ = 512, 64, 8192, 6
DOWN_T, DEPTH, DGR = 2, 3, 3

TN = 256
N_TOK = 2048


def _rvq_body(z_ref, cb_ref, cb2_ref, quant_ref, commit_ref,
              res_ref, acc_ref, comm_ref, split_ref):
    q = pl.program_id(0)
    i = pl.program_id(1)
    rows = pl.ds(i * TN, TN)

    @pl.when(q == 0)
    def _():
        res_ref[rows, :] = z_ref[...]

    cbc = cb_ref[0]

    @pl.when(i == 0)
    def _():
        c1 = cbc.astype(jnp.bfloat16)
        rem1 = cbc - c1.astype(jnp.float32)
        c2 = rem1.astype(jnp.bfloat16)
        c3 = (rem1 - c2.astype(jnp.float32)).astype(jnp.bfloat16)
        split_ref[...] = jnp.concatenate([c1, c2, c3], axis=1)

    residual = res_ref[rows, :]
    cb2 = cb2_ref[0]

    dn = (((1,), (1,)), ((), ()))
    m = jax.lax.dot_general(residual, cbc, dn,
                            preferred_element_type=jnp.float32)

    r2 = jnp.sum(residual * residual, axis=1, keepdims=True)
    dist = r2 - 2.0 * m + cb2

    dmin = jnp.min(dist, axis=1, keepdims=True)
    iota = jax.lax.broadcasted_iota(jnp.int32, (TN, NB_CODE), 1)
    idx = jnp.min(jnp.where(dist == dmin, iota, NB_CODE), axis=1,
                  keepdims=True)
    onehot = (iota == idx).astype(jnp.bfloat16)

    dg = (((1,), (0,)), ((), ()))
    g = jax.lax.dot_general(onehot, split_ref[...], dg,
                            preferred_element_type=jnp.float32)
    qv = (g[:, :CODE_DIM] + g[:, CODE_DIM:2 * CODE_DIM]) \
        + g[:, 2 * CODE_DIM:]

    new_res = residual - qv
    res_ref[rows, :] = new_res
    err = jnp.sum(new_res * new_res)

    @pl.when(q == 0)
    def _():
        acc_ref[rows, :] = qv
        comm_ref[pl.ds(i, 1), :] = jnp.full((1, 128), err, dtype=jnp.float32)

    @pl.when(q > 0)
    def _():
        acc_ref[rows, :] += qv
        comm_ref[pl.ds(i, 1), :] += jnp.full((1, 128), err, dtype=jnp.float32)

    @pl.when(q == NQ - 1)
    def _():
        quant_ref[...] = acc_ref[rows, :]
        commit_ref[...] = comm_ref[pl.ds(i, 1), :][None]


def _residual_vq(flat, codebooks):
    n, c = flat.shape
    n_tiles = n // TN
    cb2 = jnp.sum(codebooks * codebooks, axis=2)[:, None, :]
    quant, commit_parts = pl.pallas_call(
        _rvq_body,
        grid=(NQ, n_tiles),
        in_specs=[
            pl.BlockSpec((TN, c), lambda q, i: (i, 0)),
            pl.BlockSpec((1, NB_CODE, CODE_DIM), lambda q, i: (q, 0, 0)),
            pl.BlockSpec((1, 1, NB_CODE), lambda q, i: (q, 0, 0)),
        ],
        out_specs=[
            pl.BlockSpec((TN, c), lambda q, i: (i, 0)),
            pl.BlockSpec((1, 1, 128), lambda q, i: (i, 0, 0)),
        ],
        out_shape=[
            jax.ShapeDtypeStruct((n, c), jnp.float32),
            jax.ShapeDtypeStruct((n_tiles, 1, 128), jnp.float32),
        ],
        scratch_shapes=[
            pltpu.VMEM((n, c), jnp.float32),
            pltpu.VMEM((n, c), jnp.float32),
            pltpu.VMEM((n_tiles, 128), jnp.float32),
            pltpu.VMEM((NB_CODE, 3 * CODE_DIM), jnp.bfloat16),
        ],
    )(flat, codebooks, cb2)
    commit = jnp.sum(commit_parts[:, 0, 0]) / jnp.float32(n * c)
    return quant, commit


def _conv1d(x, w, b, stride=1, pad=0, dil=1):
    y = jax.lax.conv_general_dilated(
        x, w, (stride,), [(pad, pad)], rhs_dilation=(dil,),
        dimension_numbers=('NCH', 'OIH', 'NCH'))
    return y + b[None, :, None]


def _resblock(h, w1, b1, w2, b2, dil):
    r = jax.nn.relu(h)
    r = _conv1d(r, w1, b1, pad=dil, dil=dil)
    r = jax.nn.relu(r)
    r = _conv1d(r, w2, b2)
    return h + r


def _encode(h, p):
    h = jax.nn.relu(_conv1d(h, p['enc_in_w'], p['enc_in_b'], pad=1))
    for bl in range(DOWN_T):
        h = _conv1d(h, p['enc_down%d_w' % bl], p['enc_down%d_b' % bl],
                    stride=2, pad=1)
        for d in range(DEPTH):
            dil = DGR ** d
            h = _resblock(h, p['enc_res%d_%d_1_w' % (bl, d)],
                          p['enc_res%d_%d_1_b' % (bl, d)],
                          p['enc_res%d_%d_2_w' % (bl, d)],
                          p['enc_res%d_%d_2_b' % (bl, d)], dil)
    return _conv1d(h, p['enc_out_w'], p['enc_out_b'], pad=1)


def _decode(h, p):
    h = jax.nn.relu(_conv1d(h, p['dec_in_w'], p['dec_in_b'], pad=1))
    for bl in range(DOWN_T):
        for d in range(DEPTH):
            dil = DGR ** (DEPTH - 1 - d)
            h = _resblock(h, p['dec_res%d_%d_1_w' % (bl, d)],
                          p['dec_res%d_%d_1_b' % (bl, d)],
                          p['dec_res%d_%d_2_w' % (bl, d)],
                          p['dec_res%d_%d_2_b' % (bl, d)], dil)
        h = jnp.repeat(h, 2, axis=2)
        h = _conv1d(h, p['dec_up%d_w' % bl], p['dec_up%d_b' % bl], pad=1)
    h = jax.nn.relu(_conv1d(h, p['dec_out1_w'], p['dec_out1_b'], pad=1))
    return _conv1d(h, p['dec_out2_w'], p['dec_out2_b'], pad=1)


def kernel(x, codebooks, params):
    h = jnp.transpose(x, (0, 2, 1))
    z = _encode(h, params)
    b, c, t = z.shape
    flat = jnp.transpose(z, (0, 2, 1)).reshape(-1, c)
    quant, commit = _residual_vq(flat, codebooks)
    zq = jnp.transpose(quant.reshape(b, t, c), (0, 2, 1))
    out = _decode(zq, params)
    return jnp.transpose(out, (0, 2, 1)), commit


# speedup vs baseline: 1.1855x; 1.1162x over previous
import jax
import jax.numpy as jnp
from jax.experimental import pallas as pl
from jax.experimental.pallas import tpu as pltpu

B, T, IN_W = 32, 256, 263
WIDTH, CODE_DIM, NB_CODE, NQ = 512, 64, 8192, 6
DOWN_T, DEPTH, DGR = 2, 3, 3

TN = 256
N_TOK = 2048


def _quantize_tile(residual, cbc, cb2, split):
    dn = (((1,), (1,)), ((), ()))
    m = jax.lax.dot_general(residual, cbc, dn,
                            preferred_element_type=jnp.float32)

    r2 = jnp.sum(residual * residual, axis=1, keepdims=True)
    dist = r2 - 2.0 * m + cb2

    dmin = jnp.min(dist, axis=1, keepdims=True)
    iota = jax.lax.broadcasted_iota(jnp.int32, (TN, NB_CODE), 1)
    idx = jnp.min(jnp.where(dist == dmin, iota, NB_CODE), axis=1,
                  keepdims=True)
    onehot = (iota == idx).astype(jnp.bfloat16)

    dg = (((1,), (0,)), ((), ()))
    g = jax.lax.dot_general(onehot, split, dg,
                            preferred_element_type=jnp.float32)
    qv = (g[:, :CODE_DIM] + g[:, CODE_DIM:2 * CODE_DIM]) \
        + g[:, 2 * CODE_DIM:]

    new_res = residual - qv
    err = jnp.sum(new_res * new_res)
    return qv, new_res, err


def _rvq_body(z_ref, cb_ref, cb2_ref, quant_ref, commit_ref,
              res_ref, acc_ref, comm_ref, split_ref):
    q = pl.program_id(0)
    i = pl.program_id(1)
    rows0 = pl.ds(i * 2 * TN, TN)
    rows1 = pl.ds(i * 2 * TN + TN, TN)

    @pl.when(q == 0)
    def _():
        res_ref[rows0, :] = z_ref[:TN, :]
        res_ref[rows1, :] = z_ref[TN:, :]

    cbc = cb_ref[0]

    @pl.when(i == 0)
    def _():
        c1 = cbc.astype(jnp.bfloat16)
        rem1 = cbc - c1.astype(jnp.float32)
        c2 = rem1.astype(jnp.bfloat16)
        c3 = (rem1 - c2.astype(jnp.float32)).astype(jnp.bfloat16)
        split_ref[...] = jnp.concatenate([c1, c2, c3], axis=1)

    cb2 = cb2_ref[0]
    split = split_ref[...]

    qv0, nr0, err0 = _quantize_tile(res_ref[rows0, :], cbc, cb2, split)
    qv1, nr1, err1 = _quantize_tile(res_ref[rows1, :], cbc, cb2, split)
    res_ref[rows0, :] = nr0
    res_ref[rows1, :] = nr1
    err = jnp.full((1, 128), err0, dtype=jnp.float32) + err1

    @pl.when(q == 0)
    def _():
        acc_ref[rows0, :] = qv0
        acc_ref[rows1, :] = qv1
        comm_ref[pl.ds(i, 1), :] = err

    @pl.when(q > 0)
    def _():
        acc_ref[rows0, :] += qv0
        acc_ref[rows1, :] += qv1
        comm_ref[pl.ds(i, 1), :] += err

    @pl.when(q == NQ - 1)
    def _():
        quant_ref[:TN, :] = acc_ref[rows0, :]
        quant_ref[TN:, :] = acc_ref[rows1, :]
        commit_ref[...] = comm_ref[pl.ds(i, 1), :][None]


def _residual_vq(flat, codebooks):
    n, c = flat.shape
    n_pairs = n // (2 * TN)
    cb2 = jnp.sum(codebooks * codebooks, axis=2)[:, None, :]
    quant, commit_parts = pl.pallas_call(
        _rvq_body,
        grid=(NQ, n_pairs),
        in_specs=[
            pl.BlockSpec((2 * TN, c), lambda q, i: (i, 0)),
            pl.BlockSpec((1, NB_CODE, CODE_DIM), lambda q, i: (q, 0, 0)),
            pl.BlockSpec((1, 1, NB_CODE), lambda q, i: (q, 0, 0)),
        ],
        out_specs=[
            pl.BlockSpec((2 * TN, c), lambda q, i: (i, 0)),
            pl.BlockSpec((1, 1, 128), lambda q, i: (i, 0, 0)),
        ],
        out_shape=[
            jax.ShapeDtypeStruct((n, c), jnp.float32),
            jax.ShapeDtypeStruct((n_pairs, 1, 128), jnp.float32),
        ],
        scratch_shapes=[
            pltpu.VMEM((n, c), jnp.float32),
            pltpu.VMEM((n, c), jnp.float32),
            pltpu.VMEM((n_pairs, 128), jnp.float32),
            pltpu.VMEM((NB_CODE, 3 * CODE_DIM), jnp.bfloat16),
        ],
    )(flat, codebooks, cb2)
    commit = jnp.sum(commit_parts[:, 0, 0]) / jnp.float32(n * c)
    return quant, commit


def _conv1d(x, w, b, stride=1, pad=0, dil=1):
    y = jax.lax.conv_general_dilated(
        x, w, (stride,), [(pad, pad)], rhs_dilation=(dil,),
        dimension_numbers=('NCH', 'OIH', 'NCH'))
    return y + b[None, :, None]


def _resblock(h, w1, b1, w2, b2, dil):
    r = jax.nn.relu(h)
    r = _conv1d(r, w1, b1, pad=dil, dil=dil)
    r = jax.nn.relu(r)
    r = _conv1d(r, w2, b2)
    return h + r


def _encode(h, p):
    h = jax.nn.relu(_conv1d(h, p['enc_in_w'], p['enc_in_b'], pad=1))
    for bl in range(DOWN_T):
        h = _conv1d(h, p['enc_down%d_w' % bl], p['enc_down%d_b' % bl],
                    stride=2, pad=1)
        for d in range(DEPTH):
            dil = DGR ** d
            h = _resblock(h, p['enc_res%d_%d_1_w' % (bl, d)],
                          p['enc_res%d_%d_1_b' % (bl, d)],
                          p['enc_res%d_%d_2_w' % (bl, d)],
                          p['enc_res%d_%d_2_b' % (bl, d)], dil)
    return _conv1d(h, p['enc_out_w'], p['enc_out_b'], pad=1)


def _decode(h, p):
    h = jax.nn.relu(_conv1d(h, p['dec_in_w'], p['dec_in_b'], pad=1))
    for bl in range(DOWN_T):
        for d in range(DEPTH):
            dil = DGR ** (DEPTH - 1 - d)
            h = _resblock(h, p['dec_res%d_%d_1_w' % (bl, d)],
                          p['dec_res%d_%d_1_b' % (bl, d)],
                          p['dec_res%d_%d_2_w' % (bl, d)],
                          p['dec_res%d_%d_2_b' % (bl, d)], dil)
        h = jnp.repeat(h, 2, axis=2)
        h = _conv1d(h, p['dec_up%d_w' % bl], p['dec_up%d_b' % bl], pad=1)
    h = jax.nn.relu(_conv1d(h, p['dec_out1_w'], p['dec_out1_b'], pad=1))
    return _conv1d(h, p['dec_out2_w'], p['dec_out2_b'], pad=1)


def kernel(x, codebooks, params):
    h = jnp.transpose(x, (0, 2, 1))
    z = _encode(h, params)
    b, c, t = z.shape
    flat = jnp.transpose(z, (0, 2, 1)).reshape(-1, c)
    quant, commit = _residual_vq(flat, codebooks)
    zq = jnp.transpose(quant.reshape(b, t, c), (0, 2, 1))
    out = _decode(zq, params)
    return jnp.transpose(out, (0, 2, 1)), commit
